# baseline (speedup 1.0000x reference)
import functools

import jax
import jax.numpy as jnp
import numpy as np
from jax import lax
from jax.experimental import pallas as pl
from jax.experimental.pallas import tpu as pltpu
from jax.experimental.pallas import tpu_sc as plsc

_DIM = 256
_NNZ = _DIM * (_DIM + 1) // 2
_HALF = _DIM // 2
_CNT_A = _HALF * _DIM - (_HALF - 1) * _HALF // 2
_CNT_B = _NNZ - _CNT_A
_BATCH = 1024
_UNROLL = 4


def _packed_triu_idx() -> np.ndarray:
    i, j = np.triu_indices(_DIM)
    a = i < _HALF
    idx_a = (i[a] << 8) | j[a]
    idx_b = ((i[~a] - _HALF) << 8) | (j[~a] - _HALF)
    return np.concatenate([idx_a, idx_b]).astype(np.int32)


_IDX = _packed_triu_idx()


def _tri_body(x_hbm, idx_hbm, out_hbm, idx_v, buf_a, buf_b, out_v):
    nc = 2
    wid = lax.axis_index("s") * nc + lax.axis_index("c")
    per_w = _BATCH // 32
    pltpu.sync_copy(idx_hbm, idx_v)

    def gather_span(buf, chunk0, nchunks):
        def chunk_body(c, carry):
            for u in range(_UNROLL):
                base = (chunk0 + (c * _UNROLL + u)) * 16
                iv = idx_v[pl.ds(base, 16)]
                r = lax.shift_right_logical(iv, 8)
                col = lax.bitwise_and(iv, 255)
                out_v[pl.ds(base, 16)] = plsc.load_gather(buf, [r, col])
            return carry

        lax.fori_loop(0, nchunks // _UNROLL, chunk_body, 0)

    def batch_body(bl, carry):
        b = wid * per_w + bl
        pltpu.sync_copy(x_hbm.at[b, pl.ds(0, _HALF)], buf_a)
        pltpu.sync_copy(
            x_hbm.at[b, pl.ds(_HALF, _HALF), pl.ds(_HALF, _HALF)], buf_b
        )
        gather_span(buf_a, 0, _CNT_A // 16)
        gather_span(buf_b, _CNT_A // 16, _CNT_B // 16)
        pltpu.sync_copy(out_v, out_hbm.at[b])
        return carry

    lax.fori_loop(0, per_w, batch_body, 0)


@jax.jit
def _tritovec(x, idx):
    mesh = plsc.VectorSubcoreMesh(core_axis_name="c", subcore_axis_name="s")
    fn = functools.partial(
        pl.kernel,
        mesh=mesh,
        out_type=jax.ShapeDtypeStruct((_BATCH, _NNZ), jnp.float32),
        scratch_types=[
            pltpu.VMEM((_NNZ,), jnp.int32),
            pltpu.VMEM((_HALF, _DIM), jnp.float32),
            pltpu.VMEM((_HALF, _HALF), jnp.float32),
            pltpu.VMEM((_NNZ,), jnp.float32),
        ],
        compiler_params=pltpu.CompilerParams(
            use_tc_tiling_on_sc=False, needs_layout_passes=False
        ),
    )(_tri_body)
    return fn(x, idx)


def kernel(input):
    idx = jnp.asarray(_IDX)
    y = _tritovec(input, idx)
    return y[:, :, None]


# speedup vs baseline: 1.7746x; 1.7746x over previous
import functools

import jax
import jax.numpy as jnp
import numpy as np
from jax import lax
from jax.experimental import pallas as pl
from jax.experimental.pallas import tpu as pltpu
from jax.experimental.pallas import tpu_sc as plsc

_DIM = 256
_NNZ = _DIM * (_DIM + 1) // 2
_BATCH = 1024
_NTILES = 32
_PER_W = _BATCH // _NTILES
_NPH = 4
_R = _DIM // _NPH

_W = [_DIM - _R * p for p in range(_NPH)]
_CNT = [
    sum(_DIM - i for i in range(_R * p, _R * (p + 1))) for p in range(_NPH)
]
_OFF = [sum(_CNT[:p]) for p in range(_NPH)]
_CHUNKS = [c // 16 for c in _CNT]
_OBUF = max(_CNT)


def _packed_triu_idx() -> np.ndarray:
    out = []
    for p in range(_NPH):
        for i in range(_R * p, _R * (p + 1)):
            li = i - _R * p
            lj = np.arange(i, _DIM) - _R * p
            out.append((li << 8) | lj)
    idx = np.concatenate(out).astype(np.int32)
    assert idx.shape == (_NNZ,)
    return idx


_IDX = _packed_triu_idx()


def _tri_body(
    x_hbm, idx_hbm, out_hbm,
    idx_v, sbuf0, sbuf1, obuf0, obuf1,
    sem_in0, sem_in1, sem_out0, sem_out1,
):
    nc = 2
    wid = lax.axis_index("s") * nc + lax.axis_index("c")
    sbufs = (sbuf0, sbuf1)
    obufs = (obuf0, obuf1)
    sems_in = (sem_in0, sem_in1)
    sems_out = (sem_out0, sem_out1)

    pltpu.sync_copy(idx_hbm, idx_v)

    def stage_desc(b, p):
        q = p % 2
        src = x_hbm.at[b, pl.ds(_R * p, _R), pl.ds(_R * p, _W[p])]
        dst = sbufs[q].at[pl.ds(0, _R), pl.ds(0, _W[p])]
        return pltpu.make_async_copy(src, dst, sems_in[q])

    def out_desc(b, p):
        q = p % 2
        src = obufs[q].at[pl.ds(0, _CNT[p])]
        dst = out_hbm.at[b, pl.ds(_OFF[p], _CNT[p])]
        return pltpu.make_async_copy(src, dst, sems_out[q])

    def gather_phase(p):
        q = p % 2
        sb = sbufs[q]
        ob = obufs[q]

        @plsc.parallel_loop(0, _CHUNKS[p], unroll=8)
        def _(ci):
            base = _OFF[p] + ci * 16
            iv = idx_v[pl.ds(base, 16)]
            r = lax.shift_right_logical(iv, 8)
            col = lax.bitwise_and(iv, 255)
            ob[pl.ds(ci * 16, 16)] = plsc.load_gather(sb, [r, col])

    b0 = wid * _PER_W
    stage_desc(b0, 0).start()

    def batch_body(bl, carry):
        b = b0 + bl
        for p in range(_NPH):
            if p < _NPH - 1:
                stage_desc(b, p + 1).start()
            else:
                @pl.when(bl < _PER_W - 1)
                def _():
                    stage_desc(b + 1, 0).start()
            stage_desc(b, p).wait()
            if p >= 2:
                out_desc(b, p - 2).wait()
            else:
                @pl.when(bl > 0)
                def _():
                    out_desc(b - 1, p + 2).wait()
            gather_phase(p)
            out_desc(b, p).start()
        return carry

    lax.fori_loop(0, _PER_W, batch_body, 0)
    out_desc(b0 + _PER_W - 1, _NPH - 2).wait()
    out_desc(b0 + _PER_W - 1, _NPH - 1).wait()


@jax.jit
def _tritovec(x, idx):
    mesh = plsc.VectorSubcoreMesh(core_axis_name="c", subcore_axis_name="s")
    fn = functools.partial(
        pl.kernel,
        mesh=mesh,
        out_type=jax.ShapeDtypeStruct((_BATCH, _NNZ), jnp.float32),
        scratch_types=[
            pltpu.VMEM((_NNZ,), jnp.int32),
            pltpu.VMEM((_R, _DIM), jnp.float32),
            pltpu.VMEM((_R, _DIM), jnp.float32),
            pltpu.VMEM((_OBUF,), jnp.float32),
            pltpu.VMEM((_OBUF,), jnp.float32),
            pltpu.SemaphoreType.DMA,
            pltpu.SemaphoreType.DMA,
            pltpu.SemaphoreType.DMA,
            pltpu.SemaphoreType.DMA,
        ],
        compiler_params=pltpu.CompilerParams(
            use_tc_tiling_on_sc=False, needs_layout_passes=False
        ),
    )(_tri_body)
    return fn(x, idx)


def kernel(input):
    idx = jnp.asarray(_IDX)
    y = _tritovec(input, idx)
    return y[:, :, None]


# speedup vs baseline: 3.4018x; 1.9169x over previous
import functools

import jax
import jax.numpy as jnp
import numpy as np
from jax import lax
from jax.experimental import pallas as pl
from jax.experimental.pallas import tpu as pltpu
from jax.experimental.pallas import tpu_sc as plsc

_DIM = 256
_NNZ = _DIM * (_DIM + 1) // 2
_BATCH = 1024
_NTILES = 32
_PER_W = _BATCH // _NTILES
_HALF = _DIM // 2
_CNT_A = sum(_DIM - i for i in range(_HALF))
_CNT_B = _NNZ - _CNT_A


def _packed_triu_idx() -> np.ndarray:
    i, j = np.triu_indices(_DIM)
    a = i < _HALF
    idx_a = (i[a] << 8) | j[a]
    idx_b = ((i[~a] - _HALF) << 8) | (j[~a] - _HALF)
    return np.concatenate([idx_a, idx_b]).astype(np.int32)


_IDX = _packed_triu_idx()


def _tri_body(
    x_hbm, idx_hbm, out_hbm,
    idx_v, buf_a, buf_b, out_v,
    sem_a, sem_b, sem_out,
):
    nc = 2
    wid = lax.axis_index("s") * nc + lax.axis_index("c")
    b0 = wid * _PER_W

    pltpu.sync_copy(idx_hbm, idx_v)

    def stage_a(b):
        return pltpu.make_async_copy(
            x_hbm.at[b, pl.ds(0, _HALF)], buf_a, sem_a
        )

    def stage_b(b):
        return pltpu.make_async_copy(
            x_hbm.at[b, pl.ds(_HALF, _HALF), pl.ds(_HALF, _HALF)],
            buf_b,
            sem_b,
        )

    def out_copy(b):
        return pltpu.make_async_copy(
            out_v, out_hbm.at[pl.ds(b * _NNZ, _NNZ)], sem_out
        )

    def gather_span(buf, chunk0, nchunks):
        @plsc.parallel_loop(0, nchunks, unroll=8)
        def _(ci):
            base = (chunk0 + ci) * 16
            iv = idx_v[pl.ds(base, 16)]
            r = lax.shift_right_logical(iv, 8)
            col = lax.bitwise_and(iv, 255)
            out_v[pl.ds(base, 16)] = plsc.load_gather(buf, [r, col])

    stage_a(b0).start()
    stage_b(b0).start()

    def batch_body(bl, carry):
        b = b0 + bl
        stage_a(b).wait()

        @pl.when(bl > 0)
        def _():
            out_copy(b - 1).wait()

        gather_span(buf_a, 0, _CNT_A // 16)

        @pl.when(bl < _PER_W - 1)
        def _():
            stage_a(b + 1).start()

        stage_b(b).wait()
        gather_span(buf_b, _CNT_A // 16, _CNT_B // 16)

        @pl.when(bl < _PER_W - 1)
        def _():
            stage_b(b + 1).start()

        out_copy(b).start()
        return carry

    lax.fori_loop(0, _PER_W, batch_body, 0)
    out_copy(b0 + _PER_W - 1).wait()


@jax.jit
def _tritovec(x, idx):
    mesh = plsc.VectorSubcoreMesh(core_axis_name="c", subcore_axis_name="s")
    fn = functools.partial(
        pl.kernel,
        mesh=mesh,
        out_type=jax.ShapeDtypeStruct((_BATCH * _NNZ,), jnp.float32),
        scratch_types=[
            pltpu.VMEM((_NNZ,), jnp.int32),
            pltpu.VMEM((_HALF, _DIM), jnp.float32),
            pltpu.VMEM((_HALF, _HALF), jnp.float32),
            pltpu.VMEM((_NNZ,), jnp.float32),
            pltpu.SemaphoreType.DMA,
            pltpu.SemaphoreType.DMA,
            pltpu.SemaphoreType.DMA,
        ],
        compiler_params=pltpu.CompilerParams(
            use_tc_tiling_on_sc=True, needs_layout_passes=False
        ),
    )(_tri_body)
    return fn(x, idx)


def kernel(input):
    idx = jnp.asarray(_IDX)
    return _tritovec(input, idx).reshape(_BATCH, _NNZ, 1)
